# Initial kernel scaffold; baseline (speedup 1.0000x reference)
import jax
import jax.numpy as jnp
from jax.experimental import pallas as pl


def kernel(pred_box, pred_obj, gt_box, gt_obj):
    raise NotImplementedError("write your pallas kernel here")


# speedup vs baseline: 52.9165x; 52.9165x over previous
import jax
import jax.numpy as jnp
from jax.experimental import pallas as pl

_TOPK = 5


def _match_kernel(pb_ref, gt_ref, out_ref):
    pb = pb_ref[0]
    gt = gt_ref[0]
    px0 = pb[0:1, :]
    py0 = pb[1:2, :]
    px1 = pb[2:3, :]
    py1 = pb[3:4, :]
    gx0 = gt[:, 0:1]
    gy0 = gt[:, 1:2]
    gx1 = gt[:, 2:3]
    gy1 = gt[:, 3:4]

    cost_bbox = (jnp.abs(px0 - gx0) + jnp.abs(py0 - gy0)
                 + jnp.abs(px1 - gx1) + jnp.abs(py1 - gy1))

    area_p = (px1 - px0) * (py1 - py0)
    area_g = (gx1 - gx0) * (gy1 - gy0)
    lt_x = jnp.maximum(px0, gx0)
    lt_y = jnp.maximum(py0, gy0)
    rb_x = jnp.minimum(px1, gx1)
    rb_y = jnp.minimum(py1, gy1)
    wh_x = jnp.maximum(rb_x - lt_x, 0.0)
    wh_y = jnp.maximum(rb_y - lt_y, 0.0)
    inter = wh_x * wh_y
    union = area_p + area_g - inter
    iou = inter / union
    cx = jnp.maximum(px1, gx1) - jnp.minimum(px0, gx0)
    cy = jnp.maximum(py1, gy1) - jnp.minimum(py0, gy0)
    area_c = jnp.maximum(cx, 0.0) * jnp.maximum(cy, 0.0)
    giou = iou - (area_c - union) / area_c
    C = 1.0 * cost_bbox + 1.0 * (1.0 - giou)

    n_gt, n_pred = C.shape
    iot = jax.lax.broadcasted_iota(jnp.int32, (n_gt, n_pred), 1)
    cols = []
    for _ in range(_TOPK):
        m = jnp.min(C, axis=1, keepdims=True)
        imin = jnp.min(jnp.where(C == m, iot, 2**30), axis=1,
                       keepdims=True)
        cols.append(imin)
        C = jnp.where(iot == imin, jnp.inf, C)
    out_ref[0] = jnp.concatenate(cols, axis=1)


def kernel(pred_box, pred_obj, gt_box, gt_obj):
    del pred_obj, gt_obj
    B, N, _ = pred_box.shape
    M = gt_box.shape[1]
    pbT = pred_box.transpose(0, 2, 1)
    idx = pl.pallas_call(
        _match_kernel,
        grid=(B,),
        in_specs=[
            pl.BlockSpec((1, 4, N), lambda b: (b, 0, 0)),
            pl.BlockSpec((1, M, 4), lambda b: (b, 0, 0)),
        ],
        out_specs=pl.BlockSpec((1, M, _TOPK), lambda b: (b, 0, 0)),
        out_shape=jax.ShapeDtypeStruct((B, M, _TOPK), jnp.int32),
    )(pbT, gt_box)
    matched_pred = idx.reshape(B, M * _TOPK)
    matched_gt = jnp.broadcast_to(
        jnp.repeat(jnp.arange(M, dtype=jnp.int32), _TOPK), (B, M * _TOPK))
    return matched_pred, matched_gt
